# baseline (speedup 1.0000x reference)
import functools

import jax
import jax.numpy as jnp
from jax import lax
from jax.experimental import pallas as pl
from jax.experimental.pallas import tpu as pltpu
from jax.experimental.pallas import tpu_sc as plsc

D_MODEL = 2048
BANK_ROWS = 32768
HID_ROWS = 8192
NUM_CORES = 2
NUM_SUBCORES = 16
NW = NUM_CORES * NUM_SUBCORES
COLS_PER_W = D_MODEL // NW
NSL = COLS_PER_W // 16
CHUNK = 512


def _accum_chunk(buf, acc):
    def body(r, a):
        return tuple(a[c] + buf[r, pl.ds(c * 16, 16)] for c in range(NSL))

    return lax.fori_loop(0, CHUNK, body, acc, unroll=8)


def _reduce_phase(hbm, nrows, col_base, bufs, sems, acc):
    nchunks = nrows // CHUNK

    def start(i, b):
        pltpu.async_copy(
            hbm.at[pl.ds(i * CHUNK, CHUNK), pl.ds(col_base, COLS_PER_W)],
            bufs[b],
            sems[b],
        )

    def wait(b):
        pltpu.make_async_copy(
            hbm.at[pl.ds(0, CHUNK), pl.ds(col_base, COLS_PER_W)],
            bufs[b],
            sems[b],
        ).wait()

    start(0, 0)
    start(1, 1)
    npairs = nchunks // 2

    def pair_body(g, a):
        for b in range(2):
            wait(b)
            a = _accum_chunk(bufs[b], a)
            start(2 * g + b + 2, b)
        return a

    acc = lax.fori_loop(0, npairs - 1, pair_body, acc)
    for b in range(2):
        wait(b)
        acc = _accum_chunk(bufs[b], acc)
    return acc


@functools.partial(
    pl.kernel,
    out_type=jax.ShapeDtypeStruct((D_MODEL,), jnp.float32),
    mesh=plsc.VectorSubcoreMesh(core_axis_name="c", subcore_axis_name="s"),
    compiler_params=pltpu.CompilerParams(use_tc_tiling_on_sc=False),
    scratch_types=[
        pltpu.VMEM((CHUNK, COLS_PER_W), jnp.float32),
        pltpu.VMEM((CHUNK, COLS_PER_W), jnp.float32),
        pltpu.VMEM((1, COLS_PER_W), jnp.float32),
        pltpu.VMEM((COLS_PER_W,), jnp.float32),
        pltpu.SemaphoreType.DMA,
        pltpu.SemaphoreType.DMA,
    ],
)
def _titans_mean(bank_hbm, hid_hbm, out_hbm, buf0, buf1, row0, outv, sem0, sem1):
    wid = lax.axis_index("s") * NUM_CORES + lax.axis_index("c")
    col_base = pl.multiple_of(wid * COLS_PER_W, COLS_PER_W)

    zeros = jnp.zeros((16,), jnp.float32)
    acc_b = _reduce_phase(bank_hbm, BANK_ROWS, col_base,
                          (buf0, buf1), (sem0, sem1),
                          tuple(zeros for _ in range(NSL)))
    acc_h = _reduce_phase(hid_hbm, HID_ROWS, col_base,
                          (buf0, buf1), (sem0, sem1),
                          tuple(zeros for _ in range(NSL)))

    pltpu.sync_copy(
        bank_hbm.at[pl.ds(0, 1), pl.ds(col_base, COLS_PER_W)], row0
    )
    inv_bank = jnp.float32(1.0 / BANK_ROWS)
    inv_hid = jnp.float32(1.0 / HID_ROWS)
    for c in range(NSL):
        v = (acc_b[c] - row0[0, pl.ds(c * 16, 16)] + acc_h[c] * inv_hid) * inv_bank
        outv[pl.ds(c * 16, 16)] = v
    pltpu.sync_copy(outv, out_hbm.at[pl.ds(col_base, COLS_PER_W)])


def kernel(hidden, bank):
    hid2d = hidden.reshape(HID_ROWS, D_MODEL)
    return _titans_mean(bank, hid2d)


# speedup vs baseline: 2.4413x; 2.4413x over previous
import functools

import jax
import jax.numpy as jnp
from jax import lax
from jax.experimental import pallas as pl
from jax.experimental.pallas import tpu as pltpu
from jax.experimental.pallas import tpu_sc as plsc

D_MODEL = 2048
BANK_ROWS = 32768
HID_ROWS = 8192
NUM_CORES = 2
NUM_SUBCORES = 16
STRIPES = 16
COLS = D_MODEL // STRIPES
NSL = COLS // 16
CHUNK = 256


def _accum_chunk(buf, acc):
    def body(r, a):
        return tuple(a[c] + buf[r, pl.ds(c * 16, 16)] for c in range(NSL))

    return lax.fori_loop(0, CHUNK, body, acc, unroll=8)


def _reduce_phase(hbm, row_base, nrows, col_base, bufs, sems, acc):
    nchunks = nrows // CHUNK

    def start(i, b):
        pltpu.async_copy(
            hbm.at[pl.ds(row_base + i * CHUNK, CHUNK), pl.ds(col_base, COLS)],
            bufs[b],
            sems[b],
        )

    def wait(b):
        pltpu.make_async_copy(
            hbm.at[pl.ds(0, CHUNK), pl.ds(col_base, COLS)],
            bufs[b],
            sems[b],
        ).wait()

    start(0, 0)
    start(1, 1)
    npairs = nchunks // 2

    def pair_body(g, a):
        for b in range(2):
            wait(b)
            a = _accum_chunk(bufs[b], a)
            start(2 * g + b + 2, b)
        return a

    acc = lax.fori_loop(0, npairs - 1, pair_body, acc)
    for b in range(2):
        wait(b)
        acc = _accum_chunk(bufs[b], acc)
    return acc


@functools.partial(
    pl.kernel,
    out_type=jax.ShapeDtypeStruct((D_MODEL,), jnp.float32),
    mesh=plsc.VectorSubcoreMesh(core_axis_name="c", subcore_axis_name="s"),
    scratch_types=[
        pltpu.VMEM((CHUNK, COLS), jnp.float32),
        pltpu.VMEM((CHUNK, COLS), jnp.float32),
        pltpu.VMEM((8, COLS), jnp.float32),
        pltpu.VMEM((COLS,), jnp.float32),
        pltpu.VMEM((2, COLS), jnp.float32),
        pltpu.VMEM_SHARED((NUM_SUBCORES, COLS), jnp.float32),
        pltpu.SemaphoreType.DMA,
        pltpu.SemaphoreType.DMA,
    ],
)
def _titans_mean(bank_hbm, hid_hbm, out_hbm,
                 buf0, buf1, row0, partial, comb, shared, sem0, sem1):
    cid = lax.axis_index("c")
    sid = lax.axis_index("s")
    stripe = cid * (STRIPES // NUM_CORES) + sid // 2
    half = sid % 2
    col_base = pl.multiple_of(stripe * COLS, COLS)

    zeros = jnp.zeros((16,), jnp.float32)
    acc = tuple(zeros for _ in range(NSL))
    acc = _reduce_phase(bank_hbm, half * (BANK_ROWS // 2), BANK_ROWS // 2,
                        col_base, (buf0, buf1), (sem0, sem1), acc)
    acc_h = tuple(zeros for _ in range(NSL))
    acc_h = _reduce_phase(hid_hbm, half * (HID_ROWS // 2), HID_ROWS // 2,
                          col_base, (buf0, buf1), (sem0, sem1), acc_h)

    inv_hid = jnp.float32(1.0 / HID_ROWS)
    for c in range(NSL):
        partial[pl.ds(c * 16, 16)] = acc[c] + acc_h[c] * inv_hid
    pltpu.sync_copy(partial, shared.at[sid])
    plsc.subcore_barrier()

    @pl.when(half == 0)
    def _finalize():
        pltpu.sync_copy(shared.at[pl.ds(sid, 2)], comb)
        pltpu.sync_copy(bank_hbm.at[pl.ds(0, 8), pl.ds(col_base, COLS)], row0)
        inv_bank = jnp.float32(1.0 / BANK_ROWS)
        for c in range(NSL):
            s = pl.ds(c * 16, 16)
            partial[s] = (comb[0, s] + comb[1, s] - row0[0, s]) * inv_bank
        pltpu.sync_copy(partial, out_hbm.at[pl.ds(col_base, COLS)])


def kernel(hidden, bank):
    hid2d = hidden.reshape(HID_ROWS, D_MODEL)
    return _titans_mean(bank, hid2d)
